# Initial kernel scaffold; baseline (speedup 1.0000x reference)
import jax
import jax.numpy as jnp
from jax.experimental import pallas as pl


def kernel(x, global_features, Wl, bl, Wr, gamma, beta, W0, b0, W1, b1, W2, b2, edge_index, batch):
    raise NotImplementedError("write your pallas kernel here")


# speedup vs baseline: 2.1960x; 2.1960x over previous
import functools

import jax
import jax.numpy as jnp
from jax import lax
from jax.experimental import pallas as pl
from jax.experimental.pallas import tpu as pltpu
from jax.experimental.pallas import tpu_sc as plsc

N = 10000
E = 320000
D = 128
H = 128
G = 16
NG = 16
MH = 256
OUT = 1
L = 3
SLOPE = 0.042859419676898734

NC = 2
NS = 16
NW = NC * NS
CHUNK = 128
EPAD = 327680
JCH = EPAD // (NW * CHUNK)
TILE_E = EPAD // NW
NPAD = 10112
RPT = NPAD // NS


def _leaky(v):
    return jnp.where(v >= 0, v, SLOPE * v)


def _sc_agg_body(h_hbm, src_hbm, dst_hbm, zrow_hbm, part_hbm,
                 src_v, dst_v, rows_v, sem, agg_sh):
    c = lax.axis_index("c")
    s = lax.axis_index("s")
    wid = c * NS + s

    pltpu.sync_copy(src_hbm.at[pl.ds(wid * JCH, JCH)], src_v)
    pltpu.sync_copy(dst_hbm.at[pl.ds(wid * JCH, JCH)], dst_v)

    base_n = s * RPT
    pltpu.sync_copy(zrow_hbm, agg_sh.at[pl.ds(base_n, RPT)])
    plsc.subcore_barrier()

    def body(j, carry):
        pltpu.async_copy(h_hbm.at[src_v.at[j]], rows_v, sem).wait()
        pltpu.sync_copy(rows_v, agg_sh.at[dst_v.at[j]], add=True)
        return carry

    lax.fori_loop(0, JCH, body, 0)
    plsc.subcore_barrier()

    out_base = c * NPAD + base_n
    pltpu.sync_copy(agg_sh.at[pl.ds(base_n, RPT)],
                    part_hbm.at[pl.ds(out_base, RPT)])


_sc_agg = pl.kernel(
    _sc_agg_body,
    out_type=jax.ShapeDtypeStruct((NC * NPAD, D), jnp.float32),
    mesh=plsc.VectorSubcoreMesh(core_axis_name="c", subcore_axis_name="s"),
    scratch_types=[
        pltpu.VMEM((JCH, CHUNK), jnp.int32),
        pltpu.VMEM((JCH, CHUNK), jnp.int32),
        pltpu.VMEM((CHUNK, D), jnp.float32),
        pltpu.SemaphoreType.DMA,
        pltpu.VMEM_SHARED((NPAD, D), jnp.float32),
    ],
)


def _tc_layer_body(part_ref, degp_ref, h_ref, wl_ref, wr_ref, bl_ref,
                   g_ref, b_ref, out_ref):
    p = part_ref[0:N, :] + part_ref[NPAD:NPAD + N, :]
    deg = degp_ref[0:N, 0:1] + degp_ref[NPAD:NPAD + N, 0:1]
    deg = jnp.maximum(deg, 1.0)
    agg = p / deg
    hpre = (jnp.dot(agg, wl_ref[...], preferred_element_type=jnp.float32)
            + jnp.dot(h_ref[...], wr_ref[...], preferred_element_type=jnp.float32)
            + bl_ref[...])
    mean = jnp.mean(hpre, axis=0, keepdims=True)
    var = jnp.mean((hpre - mean) ** 2, axis=0, keepdims=True)
    hn = (hpre - mean) * lax.rsqrt(var + 1e-5) * g_ref[...] + b_ref[...]
    out_ref[...] = _leaky(hn)


def _tc_layer(part, degp, h, wl, wr, bl, g, b):
    return pl.pallas_call(
        _tc_layer_body,
        out_shape=jax.ShapeDtypeStruct((N, H), jnp.float32),
    )(part, degp, h, wl, wr, bl, g, b)


def _tc_head_body(h_ref, batch_ref, gf_ref, w0a_ref, w0b_ref, b0_ref,
                  w1_ref, b1_ref, w2_ref, b2_ref, out_ref):
    iota = lax.broadcasted_iota(jnp.int32, (NG, 1), 0)
    m = (batch_ref[...] == iota).astype(jnp.float32)
    pooled_sum = jnp.dot(m, h_ref[...], preferred_element_type=jnp.float32)
    cnt = jnp.dot(m, jnp.ones((N, 1), jnp.float32),
                  preferred_element_type=jnp.float32)
    pooled = pooled_sum / jnp.maximum(cnt, 1.0)
    z = _leaky(jnp.dot(pooled, w0a_ref[...], preferred_element_type=jnp.float32)
               + jnp.dot(gf_ref[...], w0b_ref[...], preferred_element_type=jnp.float32)
               + b0_ref[...])
    z = _leaky(jnp.dot(z, w1_ref[...], preferred_element_type=jnp.float32)
               + b1_ref[...])
    out_ref[...] = jnp.dot(z, w2_ref[...],
                           preferred_element_type=jnp.float32) + b2_ref[...]


def _tc_head(h, batch_row, gf, w0a, w0b, b0, w1, b1, w2, b2):
    return pl.pallas_call(
        _tc_head_body,
        out_shape=jax.ShapeDtypeStruct((NG, OUT), jnp.float32),
    )(h, batch_row, gf, w0a, w0b, b0, w1, b1, w2, b2)


def kernel(x, global_features, Wl, bl, Wr, gamma, beta, W0, b0, W1, b1, W2,
           b2, edge_index, batch):
    src = edge_index[0]
    dst = edge_index[1]
    pad = EPAD - E
    src2 = jnp.concatenate([src, jnp.zeros((pad,), jnp.int32)])
    dst1 = jnp.concatenate([dst, jnp.full((pad,), N, jnp.int32)])
    src2 = src2.reshape(EPAD // CHUNK, CHUNK)
    dst2 = dst1.reshape(EPAD // CHUNK, CHUNK)

    zrow = jnp.zeros((RPT, D), jnp.float32)

    h = x
    ones_tab = jnp.ones((N, D), jnp.float32)
    degp = _sc_agg(ones_tab, src2, dst2, zrow)
    for i in range(L):
        part = _sc_agg(h, src2, dst2, zrow)
        h = _tc_layer(part, degp, h,
                      Wl[i], Wr[i],
                      bl[i].reshape(1, H),
                      gamma[i].reshape(1, H),
                      beta[i].reshape(1, H))

    return _tc_head(h, batch.reshape(1, N), global_features,
                    W0[:H], W0[H:], b0.reshape(1, MH),
                    W1, b1.reshape(1, MH), W2, b2.reshape(1, OUT))


# speedup vs baseline: 3.0514x; 1.3895x over previous
import functools

import jax
import jax.numpy as jnp
from jax import lax
from jax.experimental import pallas as pl
from jax.experimental.pallas import tpu as pltpu
from jax.experimental.pallas import tpu_sc as plsc

N = 10000
E = 320000
D = 128
H = 128
G = 16
NG = 16
MH = 256
OUT = 1
L = 3
SLOPE = 0.042859419676898734

NC = 2
NS = 16
NW = NC * NS
CHUNK = 128
EPAD = 327680
JCH = EPAD // (NW * CHUNK)
TILE_E = EPAD // NW
NPAD = 10112
RPT = NPAD // NS
PH = 2
JPH = JCH // PH


def _leaky(v):
    return jnp.where(v >= 0, v, SLOPE * v)


def _sc_agg_body(h_hbm, src_hbm, dst_hbm, zrow_hbm, part_hbm,
                 src_v, dst_v, rows0, rows1, sem, agg_sh):
    c = lax.axis_index("c")
    s = lax.axis_index("s")
    wid = c * NS + s

    base_n = s * RPT
    pltpu.sync_copy(zrow_hbm, agg_sh.at[pl.ds(base_n, RPT)])
    plsc.subcore_barrier()

    for ph in range(PH):
        pltpu.sync_copy(src_hbm.at[pl.ds(wid * JCH + ph * JPH, JPH)], src_v)
        pltpu.sync_copy(dst_hbm.at[pl.ds(wid * JCH + ph * JPH, JPH)], dst_v)

        pltpu.async_copy(h_hbm.at[src_v.at[0]], rows0, sem)

        def body(j, carry):
            pltpu.make_async_copy(h_hbm.at[src_v.at[j]], rows0, sem).wait()

            @pl.when(j + 1 < JPH)
            def _():
                pltpu.async_copy(h_hbm.at[src_v.at[j + 1]], rows1, sem)

            pltpu.sync_copy(rows0, agg_sh.at[dst_v.at[j]], add=True)

            pltpu.make_async_copy(h_hbm.at[src_v.at[j + 1]], rows1, sem).wait()

            @pl.when(j + 2 < JPH)
            def _():
                pltpu.async_copy(h_hbm.at[src_v.at[j + 2]], rows0, sem)

            pltpu.sync_copy(rows1, agg_sh.at[dst_v.at[j + 1]], add=True)
            return carry

        lax.fori_loop(0, JPH // 2, lambda t, carry: body(t * 2, carry), 0)
    plsc.subcore_barrier()

    out_base = c * NPAD + base_n
    pltpu.sync_copy(agg_sh.at[pl.ds(base_n, RPT)],
                    part_hbm.at[pl.ds(out_base, RPT)])


_sc_agg = pl.kernel(
    _sc_agg_body,
    out_type=jax.ShapeDtypeStruct((NC * NPAD, D), jnp.float32),
    mesh=plsc.VectorSubcoreMesh(core_axis_name="c", subcore_axis_name="s"),
    scratch_types=[
        pltpu.VMEM((JPH, CHUNK), jnp.int32),
        pltpu.VMEM((JPH, CHUNK), jnp.int32),
        pltpu.VMEM((CHUNK, D), jnp.float32),
        pltpu.VMEM((CHUNK, D), jnp.float32),
        pltpu.SemaphoreType.DMA,
        pltpu.VMEM_SHARED((NPAD, D), jnp.float32),
    ],
)


def _sc_deg_body(dst_hbm, zrow_hbm, ones_hbm, degp_hbm,
                 dst_v, ones_v, deg_sh):
    c = lax.axis_index("c")
    s = lax.axis_index("s")
    wid = c * NS + s

    pltpu.sync_copy(dst_hbm.at[pl.ds(wid * JCH, JCH)], dst_v)
    base_n = s * RPT
    pltpu.sync_copy(zrow_hbm, deg_sh.at[pl.ds(base_n, RPT)])
    pltpu.sync_copy(ones_hbm, ones_v)
    plsc.subcore_barrier()

    def body(j, carry):
        pltpu.sync_copy(ones_v, deg_sh.at[dst_v.at[j]], add=True)
        return carry

    lax.fori_loop(0, JCH, body, 0)
    plsc.subcore_barrier()

    out_base = c * NPAD + base_n
    pltpu.sync_copy(deg_sh.at[pl.ds(base_n, RPT)],
                    degp_hbm.at[pl.ds(out_base, RPT)])


_sc_deg = pl.kernel(
    _sc_deg_body,
    out_type=jax.ShapeDtypeStruct((NC * NPAD, D), jnp.float32),
    mesh=plsc.VectorSubcoreMesh(core_axis_name="c", subcore_axis_name="s"),
    scratch_types=[
        pltpu.VMEM((JCH, CHUNK), jnp.int32),
        pltpu.VMEM((CHUNK, D), jnp.float32),
        pltpu.VMEM_SHARED((NPAD, D), jnp.float32),
    ],
)


def _tc_layer_body(part_ref, degp_ref, h_ref, wl_ref, wr_ref, bl_ref,
                   g_ref, b_ref, out_ref):
    p = part_ref[0:N, :] + part_ref[NPAD:NPAD + N, :]
    deg = degp_ref[0:N, 0:1] + degp_ref[NPAD:NPAD + N, 0:1]
    deg = jnp.maximum(deg, 1.0)
    agg = p / deg
    hpre = (jnp.dot(agg, wl_ref[...], preferred_element_type=jnp.float32)
            + jnp.dot(h_ref[...], wr_ref[...], preferred_element_type=jnp.float32)
            + bl_ref[...])
    mean = jnp.mean(hpre, axis=0, keepdims=True)
    var = jnp.mean((hpre - mean) ** 2, axis=0, keepdims=True)
    hn = (hpre - mean) * lax.rsqrt(var + 1e-5) * g_ref[...] + b_ref[...]
    out_ref[...] = _leaky(hn)


def _tc_layer(part, degp, h, wl, wr, bl, g, b):
    return pl.pallas_call(
        _tc_layer_body,
        out_shape=jax.ShapeDtypeStruct((N, H), jnp.float32),
    )(part, degp, h, wl, wr, bl, g, b)


def _tc_head_body(h_ref, batch_ref, gf_ref, w0a_ref, w0b_ref, b0_ref,
                  w1_ref, b1_ref, w2_ref, b2_ref, out_ref):
    iota = lax.broadcasted_iota(jnp.int32, (NG, 1), 0)
    m = (batch_ref[...] == iota).astype(jnp.float32)
    pooled_sum = jnp.dot(m, h_ref[...], preferred_element_type=jnp.float32)
    cnt = jnp.dot(m, jnp.ones((N, 1), jnp.float32),
                  preferred_element_type=jnp.float32)
    pooled = pooled_sum / jnp.maximum(cnt, 1.0)
    z = _leaky(jnp.dot(pooled, w0a_ref[...], preferred_element_type=jnp.float32)
               + jnp.dot(gf_ref[...], w0b_ref[...], preferred_element_type=jnp.float32)
               + b0_ref[...])
    z = _leaky(jnp.dot(z, w1_ref[...], preferred_element_type=jnp.float32)
               + b1_ref[...])
    out_ref[...] = jnp.dot(z, w2_ref[...],
                           preferred_element_type=jnp.float32) + b2_ref[...]


def _tc_head(h, batch_row, gf, w0a, w0b, b0, w1, b1, w2, b2):
    return pl.pallas_call(
        _tc_head_body,
        out_shape=jax.ShapeDtypeStruct((NG, OUT), jnp.float32),
    )(h, batch_row, gf, w0a, w0b, b0, w1, b1, w2, b2)


def kernel(x, global_features, Wl, bl, Wr, gamma, beta, W0, b0, W1, b1, W2,
           b2, edge_index, batch):
    src = edge_index[0]
    dst = edge_index[1]
    pad = EPAD - E
    src2 = jnp.concatenate([src, jnp.zeros((pad,), jnp.int32)])
    dst1 = jnp.concatenate([dst, jnp.full((pad,), N, jnp.int32)])
    src2 = src2.reshape(EPAD // CHUNK, CHUNK)
    dst2 = dst1.reshape(EPAD // CHUNK, CHUNK)

    zrow = jnp.zeros((RPT, D), jnp.float32)

    h = x
    ones_rows = jnp.ones((CHUNK, D), jnp.float32)
    degp = _sc_deg(dst2, zrow, ones_rows)
    for i in range(L):
        part = _sc_agg(h, src2, dst2, zrow)
        h = _tc_layer(part, degp, h,
                      Wl[i], Wr[i],
                      bl[i].reshape(1, H),
                      gamma[i].reshape(1, H),
                      beta[i].reshape(1, H))

    return _tc_head(h, batch.reshape(1, N), global_features,
                    W0[:H], W0[H:], b0.reshape(1, MH),
                    W1, b1.reshape(1, MH), W2, b2.reshape(1, OUT))


# speedup vs baseline: 3.0592x; 1.0026x over previous
import functools

import jax
import jax.numpy as jnp
from jax import lax
from jax.experimental import pallas as pl
from jax.experimental.pallas import tpu as pltpu
from jax.experimental.pallas import tpu_sc as plsc

N = 10000
E = 320000
D = 128
H = 128
G = 16
NG = 16
MH = 256
OUT = 1
L = 3
SLOPE = 0.042859419676898734

NC = 2
NS = 16
NW = NC * NS
CHUNK = 128
EPAD = 327680
JCH = EPAD // (NW * CHUNK)
TILE_E = EPAD // NW
NPAD = 10240
RPT = NPAD // NS
PH = 2
JPH = JCH // PH


def _leaky(v):
    return jnp.where(v >= 0, v, SLOPE * v)


def _sc_agg_body(h_hbm, src_hbm, dst_hbm, zrow_hbm, part_hbm,
                 src_v, dst_v, rows0, rows1, sem, agg_sh):
    c = lax.axis_index("c")
    s = lax.axis_index("s")
    wid = c * NS + s

    base_n = s * RPT
    pltpu.sync_copy(zrow_hbm, agg_sh.at[pl.ds(base_n, RPT)])
    plsc.subcore_barrier()

    for ph in range(PH):
        pltpu.sync_copy(src_hbm.at[pl.ds(wid * JCH + ph * JPH, JPH)], src_v)
        pltpu.sync_copy(dst_hbm.at[pl.ds(wid * JCH + ph * JPH, JPH)], dst_v)

        pltpu.async_copy(h_hbm.at[src_v.at[0]], rows0, sem)

        def body(j, carry):
            pltpu.make_async_copy(h_hbm.at[src_v.at[j]], rows0, sem).wait()

            @pl.when(j + 1 < JPH)
            def _():
                pltpu.async_copy(h_hbm.at[src_v.at[j + 1]], rows1, sem)

            pltpu.sync_copy(rows0, agg_sh.at[dst_v.at[j]], add=True)

            pltpu.make_async_copy(h_hbm.at[src_v.at[j + 1]], rows1, sem).wait()

            @pl.when(j + 2 < JPH)
            def _():
                pltpu.async_copy(h_hbm.at[src_v.at[j + 2]], rows0, sem)

            pltpu.sync_copy(rows1, agg_sh.at[dst_v.at[j + 1]], add=True)
            return carry

        lax.fori_loop(0, JPH // 2, lambda t, carry: body(t * 2, carry), 0)
    plsc.subcore_barrier()

    out_base = c * NPAD + base_n
    pltpu.sync_copy(agg_sh.at[pl.ds(base_n, RPT)],
                    part_hbm.at[pl.ds(out_base, RPT)])


def _make_sc_agg(dtype):
    return pl.kernel(
        _sc_agg_body,
        out_type=jax.ShapeDtypeStruct((NC * NPAD, D), dtype),
        mesh=plsc.VectorSubcoreMesh(core_axis_name="c", subcore_axis_name="s"),
        scratch_types=[
            pltpu.VMEM((JPH, CHUNK), jnp.int32),
            pltpu.VMEM((JPH, CHUNK), jnp.int32),
            pltpu.VMEM((CHUNK, D), dtype),
            pltpu.VMEM((CHUNK, D), dtype),
            pltpu.SemaphoreType.DMA,
            pltpu.VMEM_SHARED((NPAD, D), dtype),
        ],
    )


_sc_agg = _make_sc_agg(jnp.float32)


def _sc_deg_body(dst_hbm, zrow_hbm, ones_hbm, degp_hbm,
                 dst_v, ones_v, deg_sh):
    c = lax.axis_index("c")
    s = lax.axis_index("s")
    wid = c * NS + s

    pltpu.sync_copy(dst_hbm.at[pl.ds(wid * JCH, JCH)], dst_v)
    base_n = s * RPT
    pltpu.sync_copy(zrow_hbm, deg_sh.at[pl.ds(base_n, RPT)])
    pltpu.sync_copy(ones_hbm, ones_v)
    plsc.subcore_barrier()

    def body(j, carry):
        pltpu.sync_copy(ones_v, deg_sh.at[dst_v.at[j]], add=True)
        return carry

    lax.fori_loop(0, JCH, body, 0)
    plsc.subcore_barrier()

    out_base = c * NPAD + base_n
    pltpu.sync_copy(deg_sh.at[pl.ds(base_n, RPT)],
                    degp_hbm.at[pl.ds(out_base, RPT)])


_sc_deg = pl.kernel(
    _sc_deg_body,
    out_type=jax.ShapeDtypeStruct((NC * NPAD, D), jnp.float32),
    mesh=plsc.VectorSubcoreMesh(core_axis_name="c", subcore_axis_name="s"),
    scratch_types=[
        pltpu.VMEM((JCH, CHUNK), jnp.int32),
        pltpu.VMEM((CHUNK, D), jnp.float32),
        pltpu.VMEM_SHARED((NPAD, D), jnp.float32),
    ],
)


def _tc_layer_body(part_ref, degp_ref, h_ref, wl_ref, wr_ref, bl_ref,
                   g_ref, b_ref, out_ref):
    p = (part_ref[0:N, :].astype(jnp.float32)
         + part_ref[NPAD:NPAD + N, :].astype(jnp.float32))
    deg = degp_ref[0:N, 0:1] + degp_ref[NPAD:NPAD + N, 0:1]
    deg = jnp.maximum(deg, 1.0)
    agg = p / deg
    hpre = (jnp.dot(agg, wl_ref[...], preferred_element_type=jnp.float32)
            + jnp.dot(h_ref[...], wr_ref[...], preferred_element_type=jnp.float32)
            + bl_ref[...])
    mean = jnp.mean(hpre, axis=0, keepdims=True)
    var = jnp.mean((hpre - mean) ** 2, axis=0, keepdims=True)
    hn = (hpre - mean) * lax.rsqrt(var + 1e-5) * g_ref[...] + b_ref[...]
    out_ref[...] = _leaky(hn)


def _tc_layer(part, degp, h, wl, wr, bl, g, b):
    return pl.pallas_call(
        _tc_layer_body,
        out_shape=jax.ShapeDtypeStruct((N, H), jnp.float32),
    )(part, degp, h, wl, wr, bl, g, b)


def _tc_last_body(part_ref, degp_ref, h_ref, wl_ref, wr_ref, bl_ref,
                  g_ref, b_ref, batch_ref, gf_ref, w0a_ref, w0b_ref, b0_ref,
                  w1_ref, b1_ref, w2_ref, b2_ref, out_ref):
    p = (part_ref[0:N, :].astype(jnp.float32)
         + part_ref[NPAD:NPAD + N, :].astype(jnp.float32))
    deg = degp_ref[0:N, 0:1] + degp_ref[NPAD:NPAD + N, 0:1]
    deg = jnp.maximum(deg, 1.0)
    agg = p / deg
    hpre = (jnp.dot(agg, wl_ref[...], preferred_element_type=jnp.float32)
            + jnp.dot(h_ref[...], wr_ref[...], preferred_element_type=jnp.float32)
            + bl_ref[...])
    mean = jnp.mean(hpre, axis=0, keepdims=True)
    var = jnp.mean((hpre - mean) ** 2, axis=0, keepdims=True)
    hn = (hpre - mean) * lax.rsqrt(var + 1e-5) * g_ref[...] + b_ref[...]
    hn = _leaky(hn)
    iota = lax.broadcasted_iota(jnp.int32, (NG, 1), 0)
    m = (batch_ref[...] == iota).astype(jnp.float32)
    pooled_sum = jnp.dot(m, hn, preferred_element_type=jnp.float32)
    cnt = jnp.dot(m, jnp.ones((N, 1), jnp.float32),
                  preferred_element_type=jnp.float32)
    pooled = pooled_sum / jnp.maximum(cnt, 1.0)
    z = _leaky(jnp.dot(pooled, w0a_ref[...], preferred_element_type=jnp.float32)
               + jnp.dot(gf_ref[...], w0b_ref[...], preferred_element_type=jnp.float32)
               + b0_ref[...])
    z = _leaky(jnp.dot(z, w1_ref[...], preferred_element_type=jnp.float32)
               + b1_ref[...])
    out_ref[...] = jnp.dot(z, w2_ref[...],
                           preferred_element_type=jnp.float32) + b2_ref[...]


def _tc_last(part, degp, h, wl, wr, bl, g, b, batch_row, gf, w0a, w0b, b0,
             w1, b1, w2, b2):
    return pl.pallas_call(
        _tc_last_body,
        out_shape=jax.ShapeDtypeStruct((NG, OUT), jnp.float32),
    )(part, degp, h, wl, wr, bl, g, b, batch_row, gf, w0a, w0b, b0,
      w1, b1, w2, b2)


def kernel(x, global_features, Wl, bl, Wr, gamma, beta, W0, b0, W1, b1, W2,
           b2, edge_index, batch):
    src = edge_index[0]
    dst = edge_index[1]
    pad = EPAD - E
    src2 = jnp.concatenate([src, jnp.zeros((pad,), jnp.int32)])
    dst1 = jnp.concatenate([dst, jnp.full((pad,), N, jnp.int32)])
    src2 = src2.reshape(EPAD // CHUNK, CHUNK)
    dst2 = dst1.reshape(EPAD // CHUNK, CHUNK)

    zrow = jnp.zeros((RPT, D), jnp.float32)

    h = x
    ones_rows = jnp.ones((CHUNK, D), jnp.float32)
    degp = _sc_deg(dst2, zrow, ones_rows)
    for i in range(L - 1):
        part = _sc_agg(h, src2, dst2, zrow)
        h = _tc_layer(part, degp, h,
                      Wl[i], Wr[i],
                      bl[i].reshape(1, H),
                      gamma[i].reshape(1, H),
                      beta[i].reshape(1, H))

    part = _sc_agg(h, src2, dst2, zrow)
    return _tc_last(part, degp, h,
                    Wl[L - 1], Wr[L - 1],
                    bl[L - 1].reshape(1, H),
                    gamma[L - 1].reshape(1, H),
                    beta[L - 1].reshape(1, H),
                    batch.reshape(1, N), global_features,
                    W0[:H], W0[H:], b0.reshape(1, MH),
                    W1, b1.reshape(1, MH), W2, b2.reshape(1, OUT))


# speedup vs baseline: 3.0593x; 1.0000x over previous
import jax
import jax.numpy as jnp
from jax import lax
from jax.experimental import pallas as pl
from jax.experimental.pallas import tpu as pltpu
from jax.experimental.pallas import tpu_sc as plsc

N = 10000
E = 320000
D = 128
H = 128
G = 16
NG = 16
MH = 256
OUT = 1
L = 3
SLOPE = 0.042859419676898734

NC = 2
NS = 16
NW = NC * NS
CHUNK = 128
EPAD = 327680
JCH = EPAD // (NW * CHUNK)
TILE_E = EPAD // NW
NPAD = 10240
RPT = NPAD // NS
PH = 2
JPH = JCH // PH


def _leaky(v):
    return jnp.where(v >= 0, v, SLOPE * v)


def _sc_agg_body(h_hbm, src_hbm, dst_hbm, zrow_hbm, part_hbm,
                 src_v, dst_v, rows0, rows1, sem, agg_sh):
    c = lax.axis_index("c")
    s = lax.axis_index("s")
    wid = c * NS + s

    base_n = s * RPT
    pltpu.sync_copy(zrow_hbm, agg_sh.at[pl.ds(base_n, RPT)])
    plsc.subcore_barrier()

    for ph in range(PH):
        pltpu.sync_copy(src_hbm.at[pl.ds(wid * JCH + ph * JPH, JPH)], src_v)
        pltpu.sync_copy(dst_hbm.at[pl.ds(wid * JCH + ph * JPH, JPH)], dst_v)

        pltpu.async_copy(h_hbm.at[src_v.at[0]], rows0, sem)

        def body(j, carry):
            pltpu.make_async_copy(h_hbm.at[src_v.at[j]], rows0, sem).wait()

            @pl.when(j + 1 < JPH)
            def _():
                pltpu.async_copy(h_hbm.at[src_v.at[j + 1]], rows1, sem)

            pltpu.sync_copy(rows0, agg_sh.at[dst_v.at[j]], add=True)

            pltpu.make_async_copy(h_hbm.at[src_v.at[j + 1]], rows1, sem).wait()

            @pl.when(j + 2 < JPH)
            def _():
                pltpu.async_copy(h_hbm.at[src_v.at[j + 2]], rows0, sem)

            pltpu.sync_copy(rows1, agg_sh.at[dst_v.at[j + 1]], add=True)
            return carry

        lax.fori_loop(0, JPH // 2, lambda t, carry: body(t * 2, carry), 0)
    plsc.subcore_barrier()

    out_base = c * NPAD + base_n
    pltpu.sync_copy(agg_sh.at[pl.ds(base_n, RPT)],
                    part_hbm.at[pl.ds(out_base, RPT)])


_sc_agg = pl.kernel(
    _sc_agg_body,
    out_type=jax.ShapeDtypeStruct((NC * NPAD, D), jnp.float32),
    mesh=plsc.VectorSubcoreMesh(core_axis_name="c", subcore_axis_name="s"),
    scratch_types=[
        pltpu.VMEM((JPH, CHUNK), jnp.int32),
        pltpu.VMEM((JPH, CHUNK), jnp.int32),
        pltpu.VMEM((CHUNK, D), jnp.float32),
        pltpu.VMEM((CHUNK, D), jnp.float32),
        pltpu.SemaphoreType.DMA,
        pltpu.VMEM_SHARED((NPAD, D), jnp.float32),
    ],
)


def _sc_deg_body(dst_hbm, zrow_hbm, ones_hbm, degp_hbm,
                 dst_v, ones_v, deg_sh):
    c = lax.axis_index("c")
    s = lax.axis_index("s")
    wid = c * NS + s

    pltpu.sync_copy(dst_hbm.at[pl.ds(wid * JCH, JCH)], dst_v)
    base_n = s * RPT
    pltpu.sync_copy(zrow_hbm, deg_sh.at[pl.ds(base_n, RPT)])
    pltpu.sync_copy(ones_hbm, ones_v)
    plsc.subcore_barrier()

    def body(j, carry):
        pltpu.sync_copy(ones_v, deg_sh.at[dst_v.at[j]], add=True)
        return carry

    lax.fori_loop(0, JCH, body, 0)
    plsc.subcore_barrier()

    out_base = c * NPAD + base_n
    pltpu.sync_copy(deg_sh.at[pl.ds(base_n, RPT)],
                    degp_hbm.at[pl.ds(out_base, RPT)])


_sc_deg = pl.kernel(
    _sc_deg_body,
    out_type=jax.ShapeDtypeStruct((NC * NPAD, D), jnp.float32),
    mesh=plsc.VectorSubcoreMesh(core_axis_name="c", subcore_axis_name="s"),
    scratch_types=[
        pltpu.VMEM((JCH, CHUNK), jnp.int32),
        pltpu.VMEM((CHUNK, D), jnp.float32),
        pltpu.VMEM_SHARED((NPAD, D), jnp.float32),
    ],
)


def _layer_update(part_ref, degp_ref, h_ref, wl_ref, wr_ref, bl_ref,
                  g_ref, b_ref):
    p = part_ref[0:N, :] + part_ref[NPAD:NPAD + N, :]
    deg = degp_ref[0:N, 0:1] + degp_ref[NPAD:NPAD + N, 0:1]
    deg = jnp.maximum(deg, 1.0)
    agg = p / deg
    hpre = (jnp.dot(agg, wl_ref[...], preferred_element_type=jnp.float32)
            + jnp.dot(h_ref[...], wr_ref[...], preferred_element_type=jnp.float32)
            + bl_ref[...])
    mean = jnp.mean(hpre, axis=0, keepdims=True)
    var = jnp.mean((hpre - mean) ** 2, axis=0, keepdims=True)
    hn = (hpre - mean) * lax.rsqrt(var + 1e-5) * g_ref[...] + b_ref[...]
    return _leaky(hn)


def _tc_layer_body(part_ref, degp_ref, h_ref, wl_ref, wr_ref, bl_ref,
                   g_ref, b_ref, out_ref):
    out_ref[...] = _layer_update(part_ref, degp_ref, h_ref, wl_ref, wr_ref,
                                 bl_ref, g_ref, b_ref)


def _tc_layer(part, degp, h, wl, wr, bl, g, b):
    return pl.pallas_call(
        _tc_layer_body,
        out_shape=jax.ShapeDtypeStruct((N, H), jnp.float32),
    )(part, degp, h, wl, wr, bl, g, b)


def _tc_last_body(part_ref, degp_ref, h_ref, wl_ref, wr_ref, bl_ref,
                  g_ref, b_ref, batch_ref, gf_ref, w0a_ref, w0b_ref, b0_ref,
                  w1_ref, b1_ref, w2_ref, b2_ref, out_ref):
    hn = _layer_update(part_ref, degp_ref, h_ref, wl_ref, wr_ref,
                       bl_ref, g_ref, b_ref)
    iota = lax.broadcasted_iota(jnp.int32, (NG, 1), 0)
    m = (batch_ref[...] == iota).astype(jnp.float32)
    pooled_sum = jnp.dot(m, hn, preferred_element_type=jnp.float32)
    cnt = jnp.dot(m, jnp.ones((N, 1), jnp.float32),
                  preferred_element_type=jnp.float32)
    pooled = pooled_sum / jnp.maximum(cnt, 1.0)
    z = _leaky(jnp.dot(pooled, w0a_ref[...], preferred_element_type=jnp.float32)
               + jnp.dot(gf_ref[...], w0b_ref[...], preferred_element_type=jnp.float32)
               + b0_ref[...])
    z = _leaky(jnp.dot(z, w1_ref[...], preferred_element_type=jnp.float32)
               + b1_ref[...])
    out_ref[...] = jnp.dot(z, w2_ref[...],
                           preferred_element_type=jnp.float32) + b2_ref[...]


def _tc_last(part, degp, h, wl, wr, bl, g, b, batch_row, gf, w0a, w0b, b0,
             w1, b1, w2, b2):
    return pl.pallas_call(
        _tc_last_body,
        out_shape=jax.ShapeDtypeStruct((NG, OUT), jnp.float32),
    )(part, degp, h, wl, wr, bl, g, b, batch_row, gf, w0a, w0b, b0,
      w1, b1, w2, b2)


def kernel(x, global_features, Wl, bl, Wr, gamma, beta, W0, b0, W1, b1, W2,
           b2, edge_index, batch):
    src = edge_index[0]
    dst = edge_index[1]
    pad = EPAD - E
    src2 = jnp.concatenate([src, jnp.zeros((pad,), jnp.int32)])
    dst1 = jnp.concatenate([dst, jnp.full((pad,), N, jnp.int32)])
    src2 = src2.reshape(EPAD // CHUNK, CHUNK)
    dst2 = dst1.reshape(EPAD // CHUNK, CHUNK)

    zrow = jnp.zeros((RPT, D), jnp.float32)
    ones_rows = jnp.ones((CHUNK, D), jnp.float32)

    h = x
    degp = _sc_deg(dst2, zrow, ones_rows)
    for i in range(L - 1):
        part = _sc_agg(h, src2, dst2, zrow)
        h = _tc_layer(part, degp, h,
                      Wl[i], Wr[i],
                      bl[i].reshape(1, H),
                      gamma[i].reshape(1, H),
                      beta[i].reshape(1, H))

    part = _sc_agg(h, src2, dst2, zrow)
    return _tc_last(part, degp, h,
                    Wl[L - 1], Wr[L - 1],
                    bl[L - 1].reshape(1, H),
                    gamma[L - 1].reshape(1, H),
                    beta[L - 1].reshape(1, H),
                    batch.reshape(1, N), global_features,
                    W0[:H], W0[H:], b0.reshape(1, MH),
                    W1, b1.reshape(1, MH), W2, b2.reshape(1, OUT))
